# Initial kernel scaffold; baseline (speedup 1.0000x reference)
import jax
import jax.numpy as jnp
from jax.experimental import pallas as pl


def kernel(x_lc, batch_lc, enc_W1, enc_b1, enc_W2, enc_b2, conv1_W, conv1_b, conv2_W, conv2_b, conv3_W, conv3_b, out_W1, out_b1, out_W2, out_b2, out_W3, out_b3):
    raise NotImplementedError("write your pallas kernel here")


# speedup vs baseline: 4.1626x; 4.1626x over previous
import functools

import jax
import jax.numpy as jnp
from jax import lax
from jax.experimental import pallas as pl
from jax.experimental.pallas import tpu as pltpu

N_PAD = 10240
R = 128
C = 2048
K = 24
BIG = 1e9
F32 = jnp.float32
E_TOT = K * N_PAD


def _elu(x):
    return jnp.where(x > 0, x, jnp.exp(jnp.minimum(x, 0.0)) - 1.0)


def _mm_body(x_ref, w_ref, b_ref, o_ref):
    o_ref[...] = jnp.dot(x_ref[...], w_ref[...],
                         preferred_element_type=F32) + b_ref[...]


def _mm(x, w, b):
    return pl.pallas_call(
        _mm_body,
        out_shape=jax.ShapeDtypeStruct((x.shape[0], w.shape[1]), F32),
    )(x, w, b.reshape(1, -1))


def _head_body(h_ref, w1_ref, b1_ref, w2_ref, b2_ref, w3_ref, b3_ref, o_ref):
    o = _elu(jnp.dot(h_ref[...], w1_ref[...], preferred_element_type=F32)
             + b1_ref[...])
    o = _elu(jnp.dot(o, w2_ref[...], preferred_element_type=F32) + b2_ref[...])
    o_ref[...] = jnp.dot(o, w3_ref[...], preferred_element_type=F32) + b3_ref[...]


def _sel_body(hrow_ref, sqr_ref, brow_ref, h_ref, sqc_ref, bcol_ref,
              idx_ref, dbuf_ref):
    hr = hrow_ref[...]
    sqr = sqr_ref[...]
    br = brow_ref[...]
    n_chunks = N_PAD // C
    for c in range(n_chunks):
        sl = pl.ds(c * C, C)
        hc = h_ref[sl, :]
        dot = lax.dot_general(hr, hc, (((1,), (1,)), ((), ())),
                              preferred_element_type=F32)
        bc = bcol_ref[:, sl]
        d = (sqr + sqc_ref[:, sl]) - 2.0 * dot
        d = d + jnp.where(br != bc, BIG, 0.0)
        d = d + jnp.where(bc < 0, BIG, 0.0)
        dbuf_ref[:, sl] = d

    iota_out = lax.broadcasted_iota(jnp.int32, (R, 32), 1)

    def extract(e, acc):
        m = jnp.full((R, 1), jnp.inf, F32)
        for c in range(n_chunks):
            sl = pl.ds(c * C, C)
            m = jnp.minimum(m, jnp.min(dbuf_ref[:, sl], axis=1, keepdims=True))
        am = jnp.full((R, 1), jnp.int32(2 ** 30), jnp.int32)
        for c in range(n_chunks):
            sl = pl.ds(c * C, C)
            iot = lax.broadcasted_iota(jnp.int32, (R, C), 1) + (c * C)
            cand = jnp.where(dbuf_ref[:, sl] == m, iot, 2 ** 30)
            am = jnp.minimum(am, jnp.min(cand, axis=1, keepdims=True))
        for c in range(n_chunks):
            sl = pl.ds(c * C, C)
            iot = lax.broadcasted_iota(jnp.int32, (R, C), 1) + (c * C)
            dc = dbuf_ref[:, sl]
            dbuf_ref[:, sl] = jnp.where(iot == am, jnp.inf, dc)
        return jnp.where(iota_out == e, am, acc)

    acc = lax.fori_loop(0, K, extract, jnp.zeros((R, 32), jnp.int32))
    idx_ref[...] = acc


def _select(h, sqr, sqc, brow, bcol):
    nb = N_PAD // R
    return pl.pallas_call(
        _sel_body,
        grid=(nb,),
        in_specs=[
            pl.BlockSpec((R, 32), lambda i: (i, 0)),
            pl.BlockSpec((R, 1), lambda i: (i, 0)),
            pl.BlockSpec((R, 1), lambda i: (i, 0)),
            pl.BlockSpec((N_PAD, 32), lambda i: (0, 0)),
            pl.BlockSpec((1, N_PAD), lambda i: (0, 0)),
            pl.BlockSpec((1, N_PAD), lambda i: (0, 0)),
        ],
        out_specs=pl.BlockSpec((R, 32), lambda i: (i, 0)),
        out_shape=jax.ShapeDtypeStruct((N_PAD, 32), jnp.int32),
        scratch_shapes=[pltpu.VMEM((R, N_PAD), F32)],
    )(h, sqr, brow, h, sqc, bcol)


_SC_CACHE = {}


def _sc_gather():
    if "fn" in _SC_CACHE:
        return _SC_CACHE["fn"]
    from jax.experimental.pallas import tpu_sc as plsc
    info = plsc.get_sparse_core_info()
    nw = info.num_cores * info.num_subcores
    b_per_w = E_TOT // nw
    sub = 4
    ch = b_per_w // sub
    mesh = plsc.VectorSubcoreMesh(core_axis_name="c", subcore_axis_name="s")

    @functools.partial(
        pl.kernel, mesh=mesh,
        out_type=jax.ShapeDtypeStruct((E_TOT, 32), F32),
        scratch_types=[
            pltpu.VMEM((ch,), jnp.int32),
            pltpu.VMEM((ch, 32), F32),
            pltpu.SemaphoreType.DMA,
        ],
        compiler_params=pltpu.CompilerParams(use_tc_tiling_on_sc=False),
    )
    def gather(table_hbm, idx_hbm, out_hbm, idx_v, rows_v, sem):
        wid = lax.axis_index("s") * info.num_cores + lax.axis_index("c")
        for s in range(sub):
            base = wid * b_per_w + s * ch
            pltpu.sync_copy(idx_hbm.at[pl.ds(base, ch)], idx_v)
            pltpu.async_copy(table_hbm.at[idx_v], rows_v, sem).wait()
            pltpu.sync_copy(rows_v, out_hbm.at[pl.ds(base, ch)])

    _SC_CACHE["fn"] = gather
    return gather


def _edge_body(hrow_ref, g_ref, w_ref, b_ref, o_ref):
    hr = hrow_ref[...]
    w = w_ref[...]
    b = b_ref[...]
    zm = jnp.full((R, 32), -jnp.inf, F32)
    for k in range(K):
        gk = g_ref[k]
        feat = jnp.concatenate([hr, gk - hr], axis=1)
        z = jnp.dot(feat, w, preferred_element_type=F32) + b
        zm = jnp.maximum(zm, z)
    o_ref[...] = zm


def _edge_mlp(h, g3, conv_W, conv_b):
    nb = N_PAD // R
    return pl.pallas_call(
        _edge_body,
        grid=(nb,),
        in_specs=[
            pl.BlockSpec((R, 32), lambda i: (i, 0)),
            pl.BlockSpec((K, R, 32), lambda i: (0, i, 0)),
            pl.BlockSpec((64, 32), lambda i: (0, 0)),
            pl.BlockSpec((1, 32), lambda i: (0, 0)),
        ],
        out_specs=pl.BlockSpec((R, 32), lambda i: (i, 0)),
        out_shape=jax.ShapeDtypeStruct((N_PAD, 32), F32),
    )(h, g3, conv_W, conv_b.reshape(1, 32))


def _edgeconv(h, brow, bcol, conv_W, conv_b):
    sq = jnp.sum(h * h, axis=1)
    idx = _select(h, sq.reshape(N_PAD, 1), sq.reshape(1, N_PAD), brow, bcol)
    idx_flat = idx[:, :K].T.reshape(-1)
    g = _sc_gather()(h, idx_flat)
    g3 = g.reshape(K, N_PAD, 32)
    z = _edge_mlp(h, g3, conv_W, conv_b)
    return jax.nn.elu(z)


def kernel(x_lc, batch_lc, enc_W1, enc_b1, enc_W2, enc_b2,
           conv1_W, conv1_b, conv2_W, conv2_b, conv3_W, conv3_b,
           out_W1, out_b1, out_W2, out_b2, out_W3, out_b3):
    n = x_lc.shape[0]
    pad = N_PAD - n
    xp = jnp.pad(x_lc, ((0, pad), (0, 0)))
    bf = jnp.pad(batch_lc.astype(F32), (0, pad), constant_values=-1.0)
    brow = bf.reshape(N_PAD, 1)
    bcol = bf.reshape(1, N_PAD)

    h = jax.nn.elu(_mm(xp, enc_W1, enc_b1))
    h = jax.nn.elu(_mm(h, enc_W2, enc_b2))

    h = _edgeconv(h, brow, bcol, conv1_W, conv1_b)
    h = _edgeconv(h, brow, bcol, conv2_W, conv2_b)
    h = _edgeconv(h, brow, bcol, conv3_W, conv3_b)

    o = pl.pallas_call(
        _head_body,
        out_shape=jax.ShapeDtypeStruct((N_PAD, 8), F32),
    )(h, out_W1, out_b1.reshape(1, 32), out_W2, out_b2.reshape(1, 16),
      out_W3, out_b3.reshape(1, 8))
    return (o[:n], batch_lc)
